# Initial kernel scaffold; baseline (speedup 1.0000x reference)
import jax
import jax.numpy as jnp
from jax.experimental import pallas as pl


def kernel(static_cont_input, static_cat_input, history_cont_input, history_cat_input, future_input, W_static_cont, W_static_cat, W_history_cont, W_history_cat, W_future):
    raise NotImplementedError("write your pallas kernel here")


# speedup vs baseline: 8.8591x; 8.8591x over previous
import functools

import jax
import jax.numpy as jnp
from jax import lax
from jax.experimental import pallas as pl
from jax.experimental.pallas import tpu as pltpu
from jax.experimental.pallas import tpu_sc as plsc

B = 4096
H = 64
NC = 2
NS = 16
NW = NC * NS
C = 512

N_STATIC = B * 8
N_HIST = B * 200
N_FUT = B * 50


def _body(sc_idx, sca_idx, hc_idx, hca_idx, fu_idx,
          w_sc, w_sca, w_hc, w_hca, w_fu,
          out_s, out_h, out_f,
          idx_v, rows_v, sem):
    w = lax.axis_index("s") * NC + lax.axis_index("c")

    def do_chunk(idx_hbm, table, base, n, write):
        nsub = n // 128
        pltpu.sync_copy(idx_hbm.at[pl.ds(base, n)], idx_v.at[pl.ds(0, n)])
        copies = []
        for j in range(nsub):
            copies.append(pltpu.async_copy(
                table.at[idx_v.at[pl.ds(j * 128, 128)]],
                rows_v.at[pl.ds(j * 128, 128)], sem))
        for cp in copies:
            cp.wait()
        write(base, n)

    def run_stream(idx_hbm, table, write, total_rows):
        per_w = total_rows // NW
        base0 = w * per_w
        nfull = per_w // C
        tail = per_w - nfull * C

        def loop_body(i, _):
            do_chunk(idx_hbm, table, base0 + i * C, C, write)
            return 0

        lax.fori_loop(0, nfull, loop_body, 0)
        if tail:
            do_chunk(idx_hbm, table, base0 + nfull * C, tail, write)

    def w_interleave(out, parity):
        def write(base, n):
            pltpu.sync_copy(rows_v.at[pl.ds(0, n)],
                            out.at[pl.ds(base, n), parity])
        return write

    def w_linear(out):
        def write(base, n):
            pltpu.sync_copy(rows_v.at[pl.ds(0, n)], out.at[pl.ds(base, n)])
        return write

    run_stream(sc_idx, w_sc, w_interleave(out_s, 0), N_STATIC)
    run_stream(sca_idx, w_sca, w_interleave(out_s, 1), N_STATIC)
    run_stream(hc_idx, w_hc, w_interleave(out_h, 0), N_HIST)
    run_stream(hca_idx, w_hca, w_interleave(out_h, 1), N_HIST)
    run_stream(fu_idx, w_fu, w_linear(out_f), N_FUT)


@jax.jit
def _embed(sc_idx, sca_idx, hc_idx, hca_idx, fu_idx,
           w_sc, w_sca, w_hc, w_hca, w_fu):
    mesh = plsc.VectorSubcoreMesh(core_axis_name="c", subcore_axis_name="s",
                                  num_cores=NC, num_subcores=NS)
    return pl.kernel(
        _body,
        out_type=[
            jax.ShapeDtypeStruct((N_STATIC, 2, H), jnp.float32),
            jax.ShapeDtypeStruct((N_HIST, 2, H), jnp.float32),
            jax.ShapeDtypeStruct((N_FUT, H), jnp.float32),
        ],
        mesh=mesh,
        compiler_params=pltpu.CompilerParams(use_tc_tiling_on_sc=False),
        scratch_types=[
            pltpu.VMEM((C,), jnp.int32),
            pltpu.VMEM((C, H), jnp.float32),
            pltpu.SemaphoreType.DMA,
        ],
    )(sc_idx, sca_idx, hc_idx, hca_idx, fu_idx,
      w_sc, w_sca, w_hc, w_hca, w_fu)


def kernel(static_cont_input, static_cat_input, history_cont_input,
           history_cat_input, future_input, W_static_cont, W_static_cat,
           W_history_cont, W_history_cat, W_future):
    def prep(idx):
        return idx.astype(jnp.int32).reshape(-1)

    out_s, out_h, out_f = _embed(
        prep(static_cont_input), prep(static_cat_input),
        prep(history_cont_input), prep(history_cat_input),
        prep(future_input),
        W_static_cont, W_static_cat, W_history_cont, W_history_cat, W_future)
    return (out_s.reshape(B, 8, 2 * H),
            out_h.reshape(B, 200, 2 * H),
            out_f.reshape(B, 50, H))


# speedup vs baseline: 10.0046x; 1.1293x over previous
import functools

import jax
import jax.numpy as jnp
from jax import lax
from jax.experimental import pallas as pl
from jax.experimental.pallas import tpu as pltpu
from jax.experimental.pallas import tpu_sc as plsc

B = 4096
H = 64
NC = 2
NS = 16
NW = NC * NS
C = 512

N_STATIC = B * 8
N_HIST = B * 200
N_FUT = B * 50


def _body(sc_idx, sca_idx, hc_idx, hca_idx, fu_idx,
          w_sc, w_sca, w_hc, w_hca, w_fu,
          out_s, out_h, out_f,
          idx_v, rows_v, sg0, sg1, sw0, sw1):
    w = lax.axis_index("s") * NC + lax.axis_index("c")
    sems_g = (sg0, sg1)
    sems_w = (sw0, sw1)

    def start_chunk(idx_hbm, table, buf, base, n):
        pltpu.sync_copy(idx_hbm.at[pl.ds(base, n)], idx_v.at[buf, pl.ds(0, n)])
        for j in range(n // 128):
            pltpu.async_copy(
                table.at[idx_v.at[buf, pl.ds(j * 128, 128)]],
                rows_v.at[buf, pl.ds(j * 128, 128)], sems_g[buf])

    def wait_chunk(table, buf, n):
        pltpu.make_async_copy(table.at[pl.ds(0, n)],
                              rows_v.at[buf, pl.ds(0, n)],
                              sems_g[buf]).wait()

    def start_write(dst_fn, buf, base, n):
        pltpu.async_copy(rows_v.at[buf, pl.ds(0, n)], dst_fn(base, n),
                         sems_w[buf])

    def wait_write(dst_fn, buf, base, n):
        pltpu.make_async_copy(rows_v.at[buf, pl.ds(0, n)], dst_fn(base, n),
                              sems_w[buf]).wait()

    def run_stream(idx_hbm, table, dst_fn, per_w):
        base0 = w * per_w
        m = per_w // C
        tail = per_w - m * C

        def bofs(i):
            return base0 + i * C

        start_chunk(idx_hbm, table, 0, bofs(0), C)
        start_chunk(idx_hbm, table, 1, bofs(1), C)
        wait_chunk(table, 0, C)
        start_write(dst_fn, 0, bofs(0), C)

        def pair(k, _):
            i0 = 2 * k
            wait_write(dst_fn, 0, bofs(i0 - 2), C)
            start_chunk(idx_hbm, table, 0, bofs(i0), C)
            wait_chunk(table, 1, C)
            start_write(dst_fn, 1, bofs(i0 - 1), C)

            wait_write(dst_fn, 1, bofs(i0 - 1), C)
            start_chunk(idx_hbm, table, 1, bofs(i0 + 1), C)
            wait_chunk(table, 0, C)
            start_write(dst_fn, 0, bofs(i0), C)
            return 0

        lax.fori_loop(1, m // 2, pair, 0)

        wait_chunk(table, 1, C)
        start_write(dst_fn, 1, bofs(m - 1), C)
        if tail:
            wait_write(dst_fn, 0, bofs(m - 2), C)
            start_chunk(idx_hbm, table, 0, bofs(m), tail)
            wait_chunk(table, 0, tail)
            start_write(dst_fn, 0, bofs(m), tail)
            wait_write(dst_fn, 0, bofs(m), tail)
        else:
            wait_write(dst_fn, 0, bofs(m - 2), C)
        wait_write(dst_fn, 1, bofs(m - 1), C)

    def interleave(out, parity):
        return lambda base, n: out.at[pl.ds(base, n), parity]

    def linear(out):
        return lambda base, n: out.at[pl.ds(base, n)]

    run_stream(sc_idx, w_sc, interleave(out_s, 0), N_STATIC // NW)
    run_stream(sca_idx, w_sca, interleave(out_s, 1), N_STATIC // NW)
    run_stream(hc_idx, w_hc, interleave(out_h, 0), N_HIST // NW)
    run_stream(hca_idx, w_hca, interleave(out_h, 1), N_HIST // NW)
    run_stream(fu_idx, w_fu, linear(out_f), N_FUT // NW)


@jax.jit
def _embed(sc_idx, sca_idx, hc_idx, hca_idx, fu_idx,
           w_sc, w_sca, w_hc, w_hca, w_fu):
    mesh = plsc.VectorSubcoreMesh(core_axis_name="c", subcore_axis_name="s",
                                  num_cores=NC, num_subcores=NS)
    return pl.kernel(
        _body,
        out_type=[
            jax.ShapeDtypeStruct((N_STATIC, 2, H), jnp.float32),
            jax.ShapeDtypeStruct((N_HIST, 2, H), jnp.float32),
            jax.ShapeDtypeStruct((N_FUT, H), jnp.float32),
        ],
        mesh=mesh,
        compiler_params=pltpu.CompilerParams(use_tc_tiling_on_sc=False),
        scratch_types=[
            pltpu.VMEM((2, C), jnp.int32),
            pltpu.VMEM((2, C, H), jnp.float32),
            pltpu.SemaphoreType.DMA,
            pltpu.SemaphoreType.DMA,
            pltpu.SemaphoreType.DMA,
            pltpu.SemaphoreType.DMA,
        ],
    )(sc_idx, sca_idx, hc_idx, hca_idx, fu_idx,
      w_sc, w_sca, w_hc, w_hca, w_fu)


def kernel(static_cont_input, static_cat_input, history_cont_input,
           history_cat_input, future_input, W_static_cont, W_static_cat,
           W_history_cont, W_history_cat, W_future):
    def prep(idx):
        return idx.astype(jnp.int32).reshape(-1)

    out_s, out_h, out_f = _embed(
        prep(static_cont_input), prep(static_cat_input),
        prep(history_cont_input), prep(history_cat_input),
        prep(future_input),
        W_static_cont, W_static_cat, W_history_cont, W_history_cat, W_future)
    return (out_s.reshape(B, 8, 2 * H),
            out_h.reshape(B, 200, 2 * H),
            out_f.reshape(B, 50, H))


# speedup vs baseline: 10.3247x; 1.0320x over previous
import functools

import jax
import jax.numpy as jnp
from jax import lax
from jax.experimental import pallas as pl
from jax.experimental.pallas import tpu as pltpu
from jax.experimental.pallas import tpu_sc as plsc

B = 4096
H = 64
NC = 2
NS = 16
NW = NC * NS
C = 512

N_STATIC = B * 8
N_HIST = B * 200
N_FUT = B * 50

PER_W = (N_STATIC // NW, N_STATIC // NW, N_HIST // NW, N_HIST // NW,
         N_FUT // NW)
IDX_OFF = (0,
           PER_W[0],
           PER_W[0] + PER_W[1],
           PER_W[0] + PER_W[1] + PER_W[2],
           PER_W[0] + PER_W[1] + PER_W[2] + PER_W[3])
IDX_TOTAL = sum(PER_W)


def _body(sc_idx, sca_idx, hc_idx, hca_idx, fu_idx,
          w_sc, w_sca, w_hc, w_hca, w_fu,
          out_s, out_h, out_f,
          idx_v, rows_v, sg0, sg1, sw0, sw1, si):
    w = lax.axis_index("s") * NC + lax.axis_index("c")
    sems_g = (sg0, sg1)
    sems_w = (sw0, sw1)
    idx_streams = (sc_idx, sca_idx, hc_idx, hca_idx, fu_idx)

    for s, idx_hbm in enumerate(idx_streams):
        pltpu.async_copy(idx_hbm.at[pl.ds(w * PER_W[s], PER_W[s])],
                         idx_v.at[pl.ds(IDX_OFF[s], PER_W[s])], si)
    for s, idx_hbm in enumerate(idx_streams):
        pltpu.make_async_copy(idx_hbm.at[pl.ds(0, PER_W[s])],
                              idx_v.at[pl.ds(IDX_OFF[s], PER_W[s])],
                              si).wait()

    def start_chunk(table, off, buf, lbase, n):
        for j in range(n // 128):
            pltpu.async_copy(
                table.at[idx_v.at[pl.ds(off + lbase + j * 128, 128)]],
                rows_v.at[buf, pl.ds(j * 128, 128)], sems_g[buf])

    def wait_chunk(table, buf, n):
        pltpu.make_async_copy(table.at[pl.ds(0, n)],
                              rows_v.at[buf, pl.ds(0, n)],
                              sems_g[buf]).wait()

    def start_write(dst_fn, buf, base, n):
        pltpu.async_copy(rows_v.at[buf, pl.ds(0, n)], dst_fn(base, n),
                         sems_w[buf])

    def wait_write(dst_fn, buf, base, n):
        pltpu.make_async_copy(rows_v.at[buf, pl.ds(0, n)], dst_fn(base, n),
                              sems_w[buf]).wait()

    def run_stream(s, table, dst_fn):
        per_w = PER_W[s]
        off = IDX_OFF[s]
        base0 = w * per_w
        m = per_w // C
        tail = per_w - m * C

        def bofs(i):
            return base0 + i * C

        start_chunk(table, off, 0, 0, C)
        start_chunk(table, off, 1, C, C)
        wait_chunk(table, 0, C)
        start_write(dst_fn, 0, bofs(0), C)

        def pair(k, _):
            i0 = 2 * k
            wait_write(dst_fn, 0, bofs(i0 - 2), C)
            start_chunk(table, off, 0, i0 * C, C)
            wait_chunk(table, 1, C)
            start_write(dst_fn, 1, bofs(i0 - 1), C)

            wait_write(dst_fn, 1, bofs(i0 - 1), C)
            start_chunk(table, off, 1, (i0 + 1) * C, C)
            wait_chunk(table, 0, C)
            start_write(dst_fn, 0, bofs(i0), C)
            return 0

        lax.fori_loop(1, m // 2, pair, 0)

        wait_chunk(table, 1, C)
        start_write(dst_fn, 1, bofs(m - 1), C)
        if tail:
            wait_write(dst_fn, 0, bofs(m - 2), C)
            start_chunk(table, off, 0, m * C, tail)
            wait_chunk(table, 0, tail)
            start_write(dst_fn, 0, bofs(m), tail)
            wait_write(dst_fn, 0, bofs(m), tail)
        else:
            wait_write(dst_fn, 0, bofs(m - 2), C)
        wait_write(dst_fn, 1, bofs(m - 1), C)

    def interleave(out, parity):
        return lambda base, n: out.at[pl.ds(base, n), parity]

    def linear(out):
        return lambda base, n: out.at[pl.ds(base, n)]

    run_stream(0, w_sc, interleave(out_s, 0))
    run_stream(1, w_sca, interleave(out_s, 1))
    run_stream(2, w_hc, interleave(out_h, 0))
    run_stream(3, w_hca, interleave(out_h, 1))
    run_stream(4, w_fu, linear(out_f))


@jax.jit
def _embed(sc_idx, sca_idx, hc_idx, hca_idx, fu_idx,
           w_sc, w_sca, w_hc, w_hca, w_fu):
    mesh = plsc.VectorSubcoreMesh(core_axis_name="c", subcore_axis_name="s",
                                  num_cores=NC, num_subcores=NS)
    return pl.kernel(
        _body,
        out_type=[
            jax.ShapeDtypeStruct((N_STATIC, 2, H), jnp.float32),
            jax.ShapeDtypeStruct((N_HIST, 2, H), jnp.float32),
            jax.ShapeDtypeStruct((N_FUT, H), jnp.float32),
        ],
        mesh=mesh,
        compiler_params=pltpu.CompilerParams(use_tc_tiling_on_sc=False),
        scratch_types=[
            pltpu.VMEM((IDX_TOTAL,), jnp.int32),
            pltpu.VMEM((2, C, H), jnp.float32),
            pltpu.SemaphoreType.DMA,
            pltpu.SemaphoreType.DMA,
            pltpu.SemaphoreType.DMA,
            pltpu.SemaphoreType.DMA,
            pltpu.SemaphoreType.DMA,
        ],
    )(sc_idx, sca_idx, hc_idx, hca_idx, fu_idx,
      w_sc, w_sca, w_hc, w_hca, w_fu)


def kernel(static_cont_input, static_cat_input, history_cont_input,
           history_cat_input, future_input, W_static_cont, W_static_cat,
           W_history_cont, W_history_cat, W_future):
    def prep(idx):
        return idx.astype(jnp.int32).reshape(-1)

    out_s, out_h, out_f = _embed(
        prep(static_cont_input), prep(static_cat_input),
        prep(history_cont_input), prep(history_cat_input),
        prep(future_input),
        W_static_cont, W_static_cat, W_history_cont, W_history_cat, W_future)
    return (out_s.reshape(B, 8, 2 * H),
            out_h.reshape(B, 200, 2 * H),
            out_f.reshape(B, 50, H))
